# baseline (speedup 1.0000x reference)
import functools

import jax
import jax.numpy as jnp
from jax.experimental import pallas as pl
from jax.experimental.pallas import tpu as pltpu

_CL1_F = 32
_CL1_K = 25
_CL2_F = 64
_CL2_K = 25
_FC1_F = 512
_FC2_F = 10
_D = 10000
_FC1IN = 40000
_B = 16
_V2 = 2500

_FC1_BLK = 128


def _fc1_body(x_ref, w1_ref, b1_ref, o_ref):
    o_ref[...] = jnp.maximum(
        jax.lax.dot_general(
            x_ref[...], w1_ref[...], (((1,), (1,)), ((), ())),
            preferred_element_type=jnp.float32) + b1_ref[...], 0.0)


def _fc2_body(h_ref, w2_ref, b2_ref, o_ref):
    o_ref[...] = jax.lax.dot_general(
        h_ref[...], w2_ref[...], (((1,), (1,)), ((), ())),
        preferred_element_type=jnp.float32) + b2_ref[...]


def _fc_head(h, Wf1, bf1, Wf2, bf2):
    nj = _FC1_F // _FC1_BLK
    h1 = pl.pallas_call(
        _fc1_body,
        grid=(nj,),
        in_specs=[
            pl.BlockSpec((_B, _FC1IN), lambda j: (0, 0)),
            pl.BlockSpec((_FC1_BLK, _FC1IN), lambda j: (j, 0)),
            pl.BlockSpec((1, _FC1_BLK), lambda j: (0, j)),
        ],
        out_specs=pl.BlockSpec((_B, _FC1_BLK), lambda j: (0, j)),
        out_shape=jax.ShapeDtypeStruct((_B, _FC1_F), jnp.float32),
    )(h, Wf1, bf1.reshape(1, -1))
    return pl.pallas_call(
        _fc2_body,
        out_shape=jax.ShapeDtypeStruct((_B, _FC2_F), jnp.float32),
    )(h1, Wf2, bf2.reshape(1, -1))


def _spmm_rescaled(idx, vals, lm, y):
    g = vals[:, None] * jnp.take(y, idx[1], axis=0)
    Ly = jnp.zeros_like(y).at[idx[0]].add(g)
    return (2.0 / lm) * Ly - y


def _cheby(x, W, b, idx, vals, lm, K, Fout):
    Bb, Vv, Fin = x.shape
    x0 = jnp.transpose(x, (1, 2, 0)).reshape(Vv, Fin * Bb)
    xs = [x0]
    if K > 1:
        x1 = _spmm_rescaled(idx, vals, lm, x0)
        xs.append(x1)
    for _ in range(2, K):
        x2 = 2.0 * _spmm_rescaled(idx, vals, lm, x1) - x0
        xs.append(x2)
        x0, x1 = x1, x2
    xk = jnp.stack(xs, 0).reshape(K, Vv, Fin, Bb)
    xk = jnp.transpose(xk, (3, 1, 2, 0)).reshape(Bb * Vv, Fin * K)
    out = xk @ W.T + b
    return out.reshape(Bb, Vv, Fout)


def _pool4(x):
    Bb, Vv, F = x.shape
    return x.reshape(Bb, Vv // 4, 4, F).max(axis=2)


def kernel(x, d, L0_indices, L0_values, L2_indices, L2_values, lmax, W1, b1,
           W2, b2, Wf1, bf1, Wf2, bf2):
    h = x[:, :, None]
    h = _cheby(h, W1, b1, L0_indices, L0_values, lmax[0], _CL1_K, _CL1_F)
    h = jax.nn.relu(h)
    h = _pool4(h)
    h = _cheby(h, W2, b2, L2_indices, L2_values, lmax[1], _CL2_K, _CL2_F)
    h = jax.nn.relu(h)
    h = _pool4(h)
    h = h.reshape(-1, _FC1IN)
    return _fc_head(h, Wf1, bf1, Wf2, bf2)


# speedup vs baseline: 5.4748x; 5.4748x over previous
import functools

import jax
import jax.numpy as jnp
from jax import lax
from jax.experimental import pallas as pl
from jax.experimental.pallas import tpu as pltpu
from jax.experimental.pallas import tpu_sc as plsc

_B = 16
_D = 10000
_DP = 10240
_V2 = 2500
_V2P = 2560
_K = 25
_F1 = 32
_F2 = 64
_FC1_F = 512
_FC2_F = 10
_FC1IN = 40000

_NCH1, _CH1 = 79, 128
_ET1 = _NCH1 * _CH1
_NCH2, _CH2 = 40, 64
_ET2 = _NCH2 * _CH2

_mesh = plsc.VectorSubcoreMesh(
    core_axis_name="c", subcore_axis_name="s", num_cores=2, num_subcores=16)


def _bcast16(x):
    return jnp.broadcast_to(x, (16,))


_R1 = _DP // 16
_R1C = 128

@functools.partial(
    pl.kernel,
    out_type=jax.ShapeDtypeStruct((_K * _DP, 16), jnp.float32),
    mesh=_mesh,
    compiler_params=pltpu.CompilerParams(use_tc_tiling_on_sc=False),
    scratch_types=[
        pltpu.VMEM_SHARED((_DP, 16), jnp.float32),
        pltpu.VMEM((_NCH1, _CH1), jnp.int32),
        pltpu.VMEM((_NCH1, _CH1), jnp.int32),
        pltpu.VMEM((_ET1 // 16, 16), jnp.float32),
        pltpu.VMEM((_CH1, 16), jnp.float32),
        pltpu.VMEM((_R1C, 16), jnp.float32),
        pltpu.VMEM((_R1C, 16), jnp.float32),
        pltpu.VMEM((_R1C, 16), jnp.float32),
        pltpu.VMEM((_R1C, 16), jnp.float32),
        pltpu.VMEM((16,), jnp.float32),
        pltpu.SemaphoreType.DMA,
    ],
)
def _sc_l1(x0_hbm, srck_hbm, dst_hbm, val_hbm, s_hbm, out_hbm,
           acc, srci, dsti, vali, gbuf, aba, abc, abp, zbuf, svec, sem):
    cid = lax.axis_index("c")
    tid = lax.axis_index("s")

    @pl.when(cid == 0)
    def _body():
        pltpu.sync_copy(dst_hbm.at[tid], dsti)
        pltpu.sync_copy(val_hbm.at[tid], vali)
        pltpu.sync_copy(s_hbm, svec)

        def zrow(i, _):
            zbuf[i, :] = jnp.zeros((16,), jnp.float32)
            return 0
        lax.fori_loop(0, _R1C, zrow, 0)

        def ini(j, _):
            base = tid * _R1 + j * _R1C
            pltpu.sync_copy(x0_hbm.at[pl.ds(base, _R1C)], aba)
            pltpu.sync_copy(aba, out_hbm.at[pl.ds(base, _R1C)])
            pltpu.sync_copy(zbuf, acc.at[pl.ds(base, _R1C)])
            return 0
        lax.fori_loop(0, _R1 // _R1C, ini, 0)
        plsc.subcore_barrier()

        def hop(k, _):
            pltpu.sync_copy(srck_hbm.at[k, tid], srci)

            def chunk(j, _):
                pltpu.async_copy(out_hbm.at[srci.at[j]], gbuf, sem).wait()
                for g in range(_CH1 // 16):
                    vrow = vali[j * (_CH1 // 16) + g, :]
                    for l in range(16):
                        e = g * 16 + l
                        bc = jnp.take(vrow, jnp.full((16,), l, jnp.int32))
                        gbuf[e, :] = gbuf[e, :] * bc
                pltpu.sync_copy(gbuf, acc.at[dsti.at[j]], add=True)
                return 0
            lax.fori_loop(0, _NCH1, chunk, 0)
            plsc.subcore_barrier()

            w = lax.convert_element_type(k == 0, jnp.float32)
            sv = svec[...]
            alpha = sv * (2.0 - w)
            beta = _bcast16(w - 2.0)
            gamma = _bcast16(w - 1.0)
            pk = jnp.maximum(k - 1, 0)

            def ax(j, _):
                base = tid * _R1 + j * _R1C
                pltpu.sync_copy(acc.at[pl.ds(base, _R1C)], aba)
                pltpu.sync_copy(out_hbm.at[pl.ds(k * _DP + base, _R1C)], abc)
                pltpu.sync_copy(out_hbm.at[pl.ds(pk * _DP + base, _R1C)], abp)

                def row(i, _):
                    abp[i, :] = (alpha * aba[i, :] + beta * abc[i, :]
                                 + gamma * abp[i, :])
                    return 0
                lax.fori_loop(0, _R1C, row, 0)
                pltpu.sync_copy(abp, out_hbm.at[pl.ds((k + 1) * _DP + base, _R1C)])
                pltpu.sync_copy(zbuf, acc.at[pl.ds(base, _R1C)])
                return 0
            lax.fori_loop(0, _R1 // _R1C, ax, 0)
            plsc.subcore_barrier()
            return 0
        lax.fori_loop(0, _K - 1, hop, 0)


_R2 = _V2P // 16
_R2C = 32

@functools.partial(
    pl.kernel,
    out_type=jax.ShapeDtypeStruct((_K * 2 * _V2P, 256), jnp.float32),
    mesh=_mesh,
    compiler_params=pltpu.CompilerParams(use_tc_tiling_on_sc=False),
    scratch_types=[
        pltpu.VMEM_SHARED((_V2P, 256), jnp.float32),
        pltpu.VMEM((_NCH2, _CH2), jnp.int32),
        pltpu.VMEM((_NCH2, _CH2), jnp.int32),
        pltpu.VMEM((_ET2 // 16, 16), jnp.float32),
        pltpu.VMEM((_CH2, 256), jnp.float32),
        pltpu.VMEM((_R2C, 256), jnp.float32),
        pltpu.VMEM((_R2C, 256), jnp.float32),
        pltpu.VMEM((_R2C, 256), jnp.float32),
        pltpu.VMEM((_R2C, 256), jnp.float32),
        pltpu.VMEM((16,), jnp.float32),
        pltpu.SemaphoreType.DMA,
    ],
)
def _sc_l2(x0_hbm, srck_hbm, dst_hbm, val_hbm, s_hbm, out_hbm,
           acc, srci, dsti, vali, gbuf, aba, abc, abp, zbuf, svec, sem):
    cid = lax.axis_index("c")
    tid = lax.axis_index("s")

    pltpu.sync_copy(dst_hbm.at[tid], dsti)
    pltpu.sync_copy(val_hbm.at[tid], vali)
    pltpu.sync_copy(s_hbm, svec)

    def zrow(i, _):
        for r in range(16):
            zbuf[i, pl.ds(r * 16, 16)] = jnp.zeros((16,), jnp.float32)
        return 0
    lax.fori_loop(0, _R2C, zrow, 0)

    def ini(j, _):
        base = tid * _R2 + j * _R2C
        pltpu.sync_copy(x0_hbm.at[cid, pl.ds(base, _R2C)], aba)
        pltpu.sync_copy(aba, out_hbm.at[pl.ds(cid * _V2P + base, _R2C)])
        pltpu.sync_copy(zbuf, acc.at[pl.ds(base, _R2C)])
        return 0
    lax.fori_loop(0, _R2 // _R2C, ini, 0)
    plsc.subcore_barrier()

    def hop(k, _):
        pltpu.sync_copy(srck_hbm.at[k, cid, tid], srci)

        def chunk(j, _):
            pltpu.async_copy(out_hbm.at[srci.at[j]], gbuf, sem).wait()
            for g in range(_CH2 // 16):
                vrow = vali[j * (_CH2 // 16) + g, :]
                for l in range(16):
                    e = g * 16 + l
                    bc = jnp.take(vrow, jnp.full((16,), l, jnp.int32))
                    for r in range(16):
                        sl = pl.ds(r * 16, 16)
                        gbuf[e, sl] = gbuf[e, sl] * bc
            pltpu.sync_copy(gbuf, acc.at[dsti.at[j]], add=True)
            return 0
        lax.fori_loop(0, _NCH2, chunk, 0)
        plsc.subcore_barrier()

        w = lax.convert_element_type(k == 0, jnp.float32)
        sv = svec[...]
        alpha = sv * (2.0 - w)
        beta = _bcast16(w - 2.0)
        gamma = _bcast16(w - 1.0)
        pk = jnp.maximum(k - 1, 0)
        cb = (k * 2 + cid) * _V2P
        pb = (pk * 2 + cid) * _V2P
        nb = ((k + 1) * 2 + cid) * _V2P

        def ax(j, _):
            base = tid * _R2 + j * _R2C
            pltpu.sync_copy(acc.at[pl.ds(base, _R2C)], aba)
            pltpu.sync_copy(out_hbm.at[pl.ds(cb + base, _R2C)], abc)
            pltpu.sync_copy(out_hbm.at[pl.ds(pb + base, _R2C)], abp)

            def row(i, _):
                for r in range(16):
                    sl = pl.ds(r * 16, 16)
                    abp[i, sl] = (alpha * aba[i, sl] + beta * abc[i, sl]
                                  + gamma * abp[i, sl])
                return 0
            lax.fori_loop(0, _R2C, row, 0)
            pltpu.sync_copy(abp, out_hbm.at[pl.ds(nb + base, _R2C)])
            pltpu.sync_copy(zbuf, acc.at[pl.ds(base, _R2C)])
            return 0
        lax.fori_loop(0, _R2 // _R2C, ax, 0)
        plsc.subcore_barrier()
        return 0
    lax.fori_loop(0, _K - 1, hop, 0)


def _c1_body(xk_ref, bd_ref, b_ref, o_ref, acc_ref):
    k = pl.program_id(0)

    @pl.when(k == 0)
    def _():
        acc_ref[...] = jnp.zeros_like(acc_ref)

    acc_ref[...] += jax.lax.dot_general(
        xk_ref[0], bd_ref[0], (((1,), (0,)), ((), ())),
        preferred_element_type=jnp.float32)

    @pl.when(k == _K - 1)
    def _():
        h = jnp.maximum(acc_ref[: _D] + b_ref[...], 0.0)
        o_ref[...] = h.reshape(_V2, 4, 512).max(axis=1)


def _contract1(xk1, BD1, bias512):
    return pl.pallas_call(
        _c1_body,
        grid=(_K,),
        in_specs=[
            pl.BlockSpec((1, _DP, 16), lambda k: (k, 0, 0)),
            pl.BlockSpec((1, 16, 512), lambda k: (k, 0, 0)),
            pl.BlockSpec((1, 512), lambda k: (0, 0)),
        ],
        out_specs=pl.BlockSpec((_V2, 512), lambda k: (0, 0)),
        out_shape=jax.ShapeDtypeStruct((_V2, 512), jnp.float32),
        scratch_shapes=[pltpu.VMEM((_DP, 512), jnp.float32)],
    )(xk1, BD1, bias512)


def _c2_body(xk_ref, bd_ref, b_ref, o_ref, acc_ref):
    k = pl.program_id(1)

    @pl.when(k == 0)
    def _():
        acc_ref[...] = jnp.zeros_like(acc_ref)

    acc_ref[...] += jax.lax.dot_general(
        xk_ref[0, 0], bd_ref[0], (((1,), (0,)), ((), ())),
        preferred_element_type=jnp.float32)

    @pl.when(k == _K - 1)
    def _():
        h = jnp.maximum(acc_ref[...] + b_ref[...], 0.0)
        o_ref[0] = h.reshape(_V2P // 4, 4, 512).max(axis=1)


def _contract2(xk2, BD2, bias512):
    return pl.pallas_call(
        _c2_body,
        grid=(2, _K),
        in_specs=[
            pl.BlockSpec((1, 1, _V2P, 256), lambda c, k: (k, c, 0, 0)),
            pl.BlockSpec((1, 256, 512), lambda c, k: (k, 0, 0)),
            pl.BlockSpec((1, 512), lambda c, k: (0, 0)),
        ],
        out_specs=pl.BlockSpec((1, _V2P // 4, 512), lambda c, k: (c, 0, 0)),
        out_shape=jax.ShapeDtypeStruct((2, _V2P // 4, 512), jnp.float32),
        scratch_shapes=[pltpu.VMEM((_V2P, 512), jnp.float32)],
    )(xk2, BD2, bias512)


_FC1_BLK = 128


def _fc1_body(x_ref, w1_ref, b1_ref, o_ref):
    o_ref[...] = jnp.maximum(
        jax.lax.dot_general(
            x_ref[...], w1_ref[...], (((1,), (1,)), ((), ())),
            preferred_element_type=jnp.float32) + b1_ref[...], 0.0)


def _fc2_body(h_ref, w2_ref, b2_ref, o_ref):
    o_ref[...] = jax.lax.dot_general(
        h_ref[...], w2_ref[...], (((1,), (1,)), ((), ())),
        preferred_element_type=jnp.float32) + b2_ref[...]


def _fc_head(h, Wf1, bf1, Wf2, bf2):
    nj = _FC1_F // _FC1_BLK
    h1 = pl.pallas_call(
        _fc1_body,
        grid=(nj,),
        in_specs=[
            pl.BlockSpec((_B, _FC1IN), lambda j: (0, 0)),
            pl.BlockSpec((_FC1_BLK, _FC1IN), lambda j: (j, 0)),
            pl.BlockSpec((1, _FC1_BLK), lambda j: (0, j)),
        ],
        out_specs=pl.BlockSpec((_B, _FC1_BLK), lambda j: (0, j)),
        out_shape=jax.ShapeDtypeStruct((_B, _FC1_F), jnp.float32),
    )(h, Wf1, bf1.reshape(1, -1))
    return pl.pallas_call(
        _fc2_body,
        out_shape=jax.ShapeDtypeStruct((_B, _FC2_F), jnp.float32),
    )(h1, Wf2, bf2.reshape(1, -1))


def _shard_edges(idx, vals, n_tiles, e_per_tile, et_pad, nch, ch, v_mod):
    dst = idx[0].reshape(n_tiles, e_per_tile)
    src = idx[1].reshape(n_tiles, e_per_tile)
    val = vals.reshape(n_tiles, e_per_tile)
    npad = et_pad - e_per_tile
    pad_rows = jnp.broadcast_to(
        (jnp.arange(npad, dtype=jnp.int32) % v_mod), (n_tiles, npad))
    dst = jnp.concatenate([dst, pad_rows], axis=1).reshape(n_tiles, nch, ch)
    src = jnp.concatenate([src, pad_rows], axis=1).reshape(n_tiles, nch, ch)
    val = jnp.concatenate(
        [val, jnp.zeros((n_tiles, npad), jnp.float32)], axis=1
    ).reshape(n_tiles, et_pad // 16, 16)
    return dst, src, val


def kernel(x, d, L0_indices, L0_values, L2_indices, L2_values, lmax, W1, b1,
           W2, b2, Wf1, bf1, Wf2, bf2):
    f32 = jnp.float32
    s0 = (2.0 / lmax[0]).astype(f32)
    s1 = (2.0 / lmax[1]).astype(f32)

    x0_l1 = jnp.pad(x.T, ((0, _DP - _D), (0, 0)))
    dst1, src1, val1 = _shard_edges(
        L0_indices, L0_values, 16, 10000, _ET1, _NCH1, _CH1, _D)
    srck1 = src1[None] + (jnp.arange(_K - 1, dtype=jnp.int32) * _DP)[
        :, None, None, None]
    xk1 = _sc_l1(x0_l1, srck1, dst1, val1, jnp.full((16,), s0, f32))
    xk1 = xk1.reshape(_K, _DP, 16)

    BD1 = jnp.einsum("bc,ko->kbco", jnp.eye(16, dtype=f32),
                     W1.T).reshape(_K, 16, 512)
    p1 = _contract1(xk1, BD1, jnp.tile(b1, 16).reshape(1, 512))

    x0_l2 = jnp.pad(p1, ((0, _V2P - _V2), (0, 0))).reshape(
        _V2P, 2, 256).transpose(1, 0, 2)
    dst2, src2, val2 = _shard_edges(
        L2_indices, L2_values, 16, 2500, _ET2, _NCH2, _CH2, _V2)
    offs = ((jnp.arange(_K - 1, dtype=jnp.int32) * 2)[:, None]
            + jnp.arange(2, dtype=jnp.int32)[None, :]) * _V2P
    srck2 = src2[None, None] + offs[:, :, None, None, None]
    xk2 = _sc_l2(x0_l2, srck2, dst2, val2, jnp.full((16,), s1, f32))
    xk2 = xk2.reshape(_K, 2, _V2P, 256)

    W2r = W2.reshape(_F2, _F1, _K).transpose(2, 1, 0)
    BD2 = jnp.einsum("bc,kfo->kbfco", jnp.eye(8, dtype=f32),
                     W2r).reshape(_K, 256, 512)
    p2 = _contract2(xk2, BD2, jnp.tile(b2, 8).reshape(1, 512))

    h = p2[:, : _V2 // 4, :].reshape(2, 625, 8, 64).transpose(
        0, 2, 1, 3).reshape(_B, _FC1IN)
    return _fc_head(h, Wf1, bf1, Wf2, bf2)


# speedup vs baseline: 7.4077x; 1.3531x over previous
import functools

import jax
import jax.numpy as jnp
from jax import lax
from jax.experimental import pallas as pl
from jax.experimental.pallas import tpu as pltpu
from jax.experimental.pallas import tpu_sc as plsc

_B = 16
_D = 10000
_DP = 10240
_V2 = 2500
_V2P = 2560
_K = 25
_F1 = 32
_F2 = 64
_FC1_F = 512
_FC2_F = 10
_FC1IN = 40000

_NCH1, _CH1 = 80, 128
_ET1 = _NCH1 * _CH1
_NCH2, _CH2 = 40, 64
_ET2 = _NCH2 * _CH2

_mesh = plsc.VectorSubcoreMesh(
    core_axis_name="c", subcore_axis_name="s", num_cores=2, num_subcores=16)


def _bcast16(x):
    return jnp.broadcast_to(x, (16,))


_R1 = _DP // 16
_R1C = 128

@functools.partial(
    pl.kernel,
    out_type=jax.ShapeDtypeStruct((_K * _DP, 16), jnp.float32),
    mesh=_mesh,
    compiler_params=pltpu.CompilerParams(use_tc_tiling_on_sc=False),
    scratch_types=[
        pltpu.VMEM_SHARED((_DP, 16), jnp.float32),
        pltpu.VMEM((_NCH1, _CH1), jnp.int32),
        pltpu.VMEM((_NCH1, _CH1), jnp.int32),
        pltpu.VMEM((_ET1 // 16, 16), jnp.float32),
        pltpu.VMEM((_CH1, 16), jnp.float32),
        pltpu.VMEM((_CH1, 16), jnp.float32),
        pltpu.VMEM((_R1C, 16), jnp.float32),
        pltpu.VMEM((_R1C, 16), jnp.float32),
        pltpu.VMEM((_R1C, 16), jnp.float32),
        pltpu.VMEM((_R1C, 16), jnp.float32),
        pltpu.VMEM((16,), jnp.float32),
        pltpu.SemaphoreType.DMA,
        pltpu.SemaphoreType.DMA,
        pltpu.SemaphoreType.DMA,
        pltpu.SemaphoreType.DMA,
    ],
)
def _sc_l1(x0_hbm, srck_hbm, dst_hbm, val_hbm, s_hbm, out_hbm,
           acc, srci, dsti, vali, gbuf0, gbuf1, aba, abc, abp, zbuf, svec,
           sg0, sg1, ss0, ss1):
    cid = lax.axis_index("c")
    tid = lax.axis_index("s")

    @pl.when(cid == 0)
    def _body():
        pltpu.sync_copy(dst_hbm.at[tid], dsti)
        pltpu.sync_copy(val_hbm.at[tid], vali)
        pltpu.sync_copy(s_hbm, svec)

        def zrow(i, _):
            zbuf[i, :] = jnp.zeros((16,), jnp.float32)
            return 0
        lax.fori_loop(0, _R1C, zrow, 0)

        def ini(j, _):
            base = tid * _R1 + j * _R1C
            pltpu.sync_copy(x0_hbm.at[pl.ds(base, _R1C)], aba)
            pltpu.sync_copy(aba, out_hbm.at[pl.ds(base, _R1C)])
            pltpu.sync_copy(zbuf, acc.at[pl.ds(base, _R1C)])
            return 0
        lax.fori_loop(0, _R1 // _R1C, ini, 0)
        plsc.subcore_barrier()

        bufs = ((gbuf0, sg0, ss0), (gbuf1, sg1, ss1))

        def g_fire(j, buf, sg):
            pltpu.async_copy(out_hbm.at[srci.at[j]], buf, sg)

        def g_wait(j, buf, sg):
            pltpu.make_async_copy(out_hbm.at[srci.at[j]], buf, sg).wait()

        def s_fire(j, buf, ss):
            pltpu.async_copy(buf, acc.at[dsti.at[j]], ss, add=True)

        def s_wait(j, buf, ss):
            pltpu.make_async_copy(buf, acc.at[dsti.at[j]], ss).wait()

        def hop(k, _):
            pltpu.sync_copy(srck_hbm.at[k, tid], srci)
            g_fire(0, gbuf0, sg0)

            def pair(g, _):
                for b in range(2):
                    buf, sg, ss = bufs[b]
                    obuf, osg, oss = bufs[1 - b]
                    j = 2 * g + b
                    g_wait(j, buf, sg)

                    @pl.when(j + 1 < _NCH1)
                    def _pref():
                        @pl.when(j >= 1)
                        def _dr():
                            s_wait(j - 1, obuf, oss)
                        g_fire(j + 1, obuf, osg)

                    for gg in range(_CH1 // 16):
                        vrow = vali[j * (_CH1 // 16) + gg, :]
                        for l in range(16):
                            e = gg * 16 + l
                            bc = jnp.take(vrow, jnp.full((16,), l, jnp.int32))
                            buf[e, :] = buf[e, :] * bc
                    s_fire(j, buf, ss)
                return 0
            lax.fori_loop(0, _NCH1 // 2, pair, 0)
            s_wait(_NCH1 - 2, gbuf0, ss0)
            s_wait(_NCH1 - 1, gbuf1, ss1)
            plsc.subcore_barrier()

            w = lax.convert_element_type(k == 0, jnp.float32)
            sv = svec[...]
            alpha = sv * (2.0 - w)
            beta = _bcast16(w - 2.0)
            gamma = _bcast16(w - 1.0)
            pk = jnp.maximum(k - 1, 0)

            def ax(j, _):
                base = tid * _R1 + j * _R1C
                pltpu.sync_copy(acc.at[pl.ds(base, _R1C)], aba)
                pltpu.sync_copy(out_hbm.at[pl.ds(k * _DP + base, _R1C)], abc)
                pltpu.sync_copy(out_hbm.at[pl.ds(pk * _DP + base, _R1C)], abp)

                def row(i, _):
                    abp[i, :] = (alpha * aba[i, :] + beta * abc[i, :]
                                 + gamma * abp[i, :])
                    return 0
                lax.fori_loop(0, _R1C, row, 0)
                pltpu.sync_copy(abp, out_hbm.at[pl.ds((k + 1) * _DP + base, _R1C)])
                pltpu.sync_copy(zbuf, acc.at[pl.ds(base, _R1C)])
                return 0
            lax.fori_loop(0, _R1 // _R1C, ax, 0)
            plsc.subcore_barrier()
            return 0
        lax.fori_loop(0, _K - 1, hop, 0)


_R2 = _V2P // 16
_R2C = 32

@functools.partial(
    pl.kernel,
    out_type=jax.ShapeDtypeStruct((_K * 2 * _V2P, 256), jnp.float32),
    mesh=_mesh,
    compiler_params=pltpu.CompilerParams(use_tc_tiling_on_sc=False),
    scratch_types=[
        pltpu.VMEM_SHARED((_V2P, 256), jnp.float32),
        pltpu.VMEM((_NCH2, _CH2), jnp.int32),
        pltpu.VMEM((_NCH2, _CH2), jnp.int32),
        pltpu.VMEM((_ET2 // 16, 16), jnp.float32),
        pltpu.VMEM((_CH2, 256), jnp.float32),
        pltpu.VMEM((_CH2, 256), jnp.float32),
        pltpu.VMEM((_R2C, 256), jnp.float32),
        pltpu.VMEM((_R2C, 256), jnp.float32),
        pltpu.VMEM((_R2C, 256), jnp.float32),
        pltpu.VMEM((_R2C, 256), jnp.float32),
        pltpu.VMEM((16,), jnp.float32),
        pltpu.SemaphoreType.DMA,
        pltpu.SemaphoreType.DMA,
        pltpu.SemaphoreType.DMA,
        pltpu.SemaphoreType.DMA,
    ],
)
def _sc_l2(x0_hbm, srck_hbm, dst_hbm, val_hbm, s_hbm, out_hbm,
           acc, srci, dsti, vali, gbuf0, gbuf1, aba, abc, abp, zbuf, svec,
           sg0, sg1, ss0, ss1):
    cid = lax.axis_index("c")
    tid = lax.axis_index("s")

    pltpu.sync_copy(dst_hbm.at[tid], dsti)
    pltpu.sync_copy(val_hbm.at[tid], vali)
    pltpu.sync_copy(s_hbm, svec)

    def zrow(i, _):
        for r in range(16):
            zbuf[i, pl.ds(r * 16, 16)] = jnp.zeros((16,), jnp.float32)
        return 0
    lax.fori_loop(0, _R2C, zrow, 0)

    def ini(j, _):
        base = tid * _R2 + j * _R2C
        pltpu.sync_copy(x0_hbm.at[cid, pl.ds(base, _R2C)], aba)
        pltpu.sync_copy(aba, out_hbm.at[pl.ds(cid * _V2P + base, _R2C)])
        pltpu.sync_copy(zbuf, acc.at[pl.ds(base, _R2C)])
        return 0
    lax.fori_loop(0, _R2 // _R2C, ini, 0)
    plsc.subcore_barrier()

    bufs = ((gbuf0, sg0, ss0), (gbuf1, sg1, ss1))

    def g_fire(j, buf, sg):
        pltpu.async_copy(out_hbm.at[srci.at[j]], buf, sg)

    def g_wait(j, buf, sg):
        pltpu.make_async_copy(out_hbm.at[srci.at[j]], buf, sg).wait()

    def s_fire(j, buf, ss):
        pltpu.async_copy(buf, acc.at[dsti.at[j]], ss, add=True)

    def s_wait(j, buf, ss):
        pltpu.make_async_copy(buf, acc.at[dsti.at[j]], ss).wait()

    def hop(k, _):
        pltpu.sync_copy(srck_hbm.at[k, cid, tid], srci)
        g_fire(0, gbuf0, sg0)

        def pair(g, _):
            for b in range(2):
                buf, sg, ss = bufs[b]
                obuf, osg, oss = bufs[1 - b]
                j = 2 * g + b
                g_wait(j, buf, sg)

                @pl.when(j + 1 < _NCH2)
                def _pref():
                    @pl.when(j >= 1)
                    def _dr():
                        s_wait(j - 1, obuf, oss)
                    g_fire(j + 1, obuf, osg)

                def grp(gg, _):
                    vrow = vali[j * (_CH2 // 16) + gg, :]
                    for l in range(16):
                        bc = jnp.take(vrow, jnp.full((16,), l, jnp.int32))
                        for r in range(16):
                            sl = pl.ds(r * 16, 16)
                            buf[gg * 16 + l, sl] = buf[gg * 16 + l, sl] * bc
                    return 0
                lax.fori_loop(0, _CH2 // 16, grp, 0)
                s_fire(j, buf, ss)
            return 0
        lax.fori_loop(0, _NCH2 // 2, pair, 0)
        s_wait(_NCH2 - 2, gbuf0, ss0)
        s_wait(_NCH2 - 1, gbuf1, ss1)
        plsc.subcore_barrier()

        w = lax.convert_element_type(k == 0, jnp.float32)
        sv = svec[...]
        alpha = sv * (2.0 - w)
        beta = _bcast16(w - 2.0)
        gamma = _bcast16(w - 1.0)
        pk = jnp.maximum(k - 1, 0)
        cb = (k * 2 + cid) * _V2P
        pb = (pk * 2 + cid) * _V2P
        nb = ((k + 1) * 2 + cid) * _V2P

        def ax(j, _):
            base = tid * _R2 + j * _R2C
            pltpu.sync_copy(acc.at[pl.ds(base, _R2C)], aba)
            pltpu.sync_copy(out_hbm.at[pl.ds(cb + base, _R2C)], abc)
            pltpu.sync_copy(out_hbm.at[pl.ds(pb + base, _R2C)], abp)

            def row(i, _):
                for r in range(16):
                    sl = pl.ds(r * 16, 16)
                    abp[i, sl] = (alpha * aba[i, sl] + beta * abc[i, sl]
                                  + gamma * abp[i, sl])
                return 0
            lax.fori_loop(0, _R2C, row, 0)
            pltpu.sync_copy(abp, out_hbm.at[pl.ds(nb + base, _R2C)])
            pltpu.sync_copy(zbuf, acc.at[pl.ds(base, _R2C)])
            return 0
        lax.fori_loop(0, _R2 // _R2C, ax, 0)
        plsc.subcore_barrier()
        return 0
    lax.fori_loop(0, _K - 1, hop, 0)


def _c1_body(xk_ref, bd_ref, b_ref, o_ref, acc_ref):
    k = pl.program_id(0)

    @pl.when(k == 0)
    def _():
        acc_ref[...] = jnp.zeros_like(acc_ref)

    acc_ref[...] += jax.lax.dot_general(
        xk_ref[0], bd_ref[0], (((1,), (0,)), ((), ())),
        preferred_element_type=jnp.float32)

    @pl.when(k == _K - 1)
    def _():
        h = jnp.maximum(acc_ref[: _D] + b_ref[...], 0.0)
        o_ref[...] = h.reshape(_V2, 4, 512).max(axis=1)


def _contract1(xk1, BD1, bias512):
    return pl.pallas_call(
        _c1_body,
        grid=(_K,),
        in_specs=[
            pl.BlockSpec((1, _DP, 16), lambda k: (k, 0, 0)),
            pl.BlockSpec((1, 16, 512), lambda k: (k, 0, 0)),
            pl.BlockSpec((1, 512), lambda k: (0, 0)),
        ],
        out_specs=pl.BlockSpec((_V2, 512), lambda k: (0, 0)),
        out_shape=jax.ShapeDtypeStruct((_V2, 512), jnp.float32),
        scratch_shapes=[pltpu.VMEM((_DP, 512), jnp.float32)],
    )(xk1, BD1, bias512)


def _c2_body(xk_ref, bd_ref, b_ref, o_ref, acc_ref):
    k = pl.program_id(1)

    @pl.when(k == 0)
    def _():
        acc_ref[...] = jnp.zeros_like(acc_ref)

    acc_ref[...] += jax.lax.dot_general(
        xk_ref[0, 0], bd_ref[0], (((1,), (0,)), ((), ())),
        preferred_element_type=jnp.float32)

    @pl.when(k == _K - 1)
    def _():
        h = jnp.maximum(acc_ref[...] + b_ref[...], 0.0)
        o_ref[0] = h.reshape(_V2P // 4, 4, 512).max(axis=1)


def _contract2(xk2, BD2, bias512):
    return pl.pallas_call(
        _c2_body,
        grid=(2, _K),
        in_specs=[
            pl.BlockSpec((1, 1, _V2P, 256), lambda c, k: (k, c, 0, 0)),
            pl.BlockSpec((1, 256, 512), lambda c, k: (k, 0, 0)),
            pl.BlockSpec((1, 512), lambda c, k: (0, 0)),
        ],
        out_specs=pl.BlockSpec((1, _V2P // 4, 512), lambda c, k: (c, 0, 0)),
        out_shape=jax.ShapeDtypeStruct((2, _V2P // 4, 512), jnp.float32),
        scratch_shapes=[pltpu.VMEM((_V2P, 512), jnp.float32)],
    )(xk2, BD2, bias512)


_FC1_BLK = 128


def _fc1_body(x_ref, w1_ref, b1_ref, o_ref):
    o_ref[...] = jnp.maximum(
        jax.lax.dot_general(
            x_ref[...], w1_ref[...], (((1,), (1,)), ((), ())),
            preferred_element_type=jnp.float32) + b1_ref[...], 0.0)


def _fc2_body(h_ref, w2_ref, b2_ref, o_ref):
    o_ref[...] = jax.lax.dot_general(
        h_ref[...], w2_ref[...], (((1,), (1,)), ((), ())),
        preferred_element_type=jnp.float32) + b2_ref[...]


def _fc_head(h, Wf1, bf1, Wf2, bf2):
    nj = _FC1_F // _FC1_BLK
    h1 = pl.pallas_call(
        _fc1_body,
        grid=(nj,),
        in_specs=[
            pl.BlockSpec((_B, _FC1IN), lambda j: (0, 0)),
            pl.BlockSpec((_FC1_BLK, _FC1IN), lambda j: (j, 0)),
            pl.BlockSpec((1, _FC1_BLK), lambda j: (0, j)),
        ],
        out_specs=pl.BlockSpec((_B, _FC1_BLK), lambda j: (0, j)),
        out_shape=jax.ShapeDtypeStruct((_B, _FC1_F), jnp.float32),
    )(h, Wf1, bf1.reshape(1, -1))
    return pl.pallas_call(
        _fc2_body,
        out_shape=jax.ShapeDtypeStruct((_B, _FC2_F), jnp.float32),
    )(h1, Wf2, bf2.reshape(1, -1))


def _shard_edges(idx, vals, n_tiles, e_per_tile, et_pad, nch, ch, v_mod):
    dst = idx[0].reshape(n_tiles, e_per_tile)
    src = idx[1].reshape(n_tiles, e_per_tile)
    val = vals.reshape(n_tiles, e_per_tile)
    npad = et_pad - e_per_tile
    pad_rows = jnp.broadcast_to(
        (jnp.arange(npad, dtype=jnp.int32) % v_mod), (n_tiles, npad))
    dst = jnp.concatenate([dst, pad_rows], axis=1).reshape(n_tiles, nch, ch)
    src = jnp.concatenate([src, pad_rows], axis=1).reshape(n_tiles, nch, ch)
    val = jnp.concatenate(
        [val, jnp.zeros((n_tiles, npad), jnp.float32)], axis=1
    ).reshape(n_tiles, et_pad // 16, 16)
    return dst, src, val


def kernel(x, d, L0_indices, L0_values, L2_indices, L2_values, lmax, W1, b1,
           W2, b2, Wf1, bf1, Wf2, bf2):
    f32 = jnp.float32
    s0 = (2.0 / lmax[0]).astype(f32)
    s1 = (2.0 / lmax[1]).astype(f32)

    x0_l1 = jnp.pad(x.T, ((0, _DP - _D), (0, 0)))
    dst1, src1, val1 = _shard_edges(
        L0_indices, L0_values, 16, 10000, _ET1, _NCH1, _CH1, _D)
    srck1 = src1[None] + (jnp.arange(_K - 1, dtype=jnp.int32) * _DP)[
        :, None, None, None]
    xk1 = _sc_l1(x0_l1, srck1, dst1, val1, jnp.full((16,), s0, f32))
    xk1 = xk1.reshape(_K, _DP, 16)

    BD1 = jnp.einsum("bc,ko->kbco", jnp.eye(16, dtype=f32),
                     W1.T).reshape(_K, 16, 512)
    p1 = _contract1(xk1, BD1, jnp.tile(b1, 16).reshape(1, 512))

    x0_l2 = jnp.pad(p1, ((0, _V2P - _V2), (0, 0))).reshape(
        _V2P, 2, 256).transpose(1, 0, 2)
    dst2, src2, val2 = _shard_edges(
        L2_indices, L2_values, 16, 2500, _ET2, _NCH2, _CH2, _V2)
    offs = ((jnp.arange(_K - 1, dtype=jnp.int32) * 2)[:, None]
            + jnp.arange(2, dtype=jnp.int32)[None, :]) * _V2P
    srck2 = src2[None, None] + offs[:, :, None, None, None]
    xk2 = _sc_l2(x0_l2, srck2, dst2, val2, jnp.full((16,), s1, f32))
    xk2 = xk2.reshape(_K, 2, _V2P, 256)

    W2r = W2.reshape(_F2, _F1, _K).transpose(2, 1, 0)
    BD2 = jnp.einsum("bc,kfo->kbfco", jnp.eye(8, dtype=f32),
                     W2r).reshape(_K, 256, 512)
    p2 = _contract2(xk2, BD2, jnp.tile(b2, 8).reshape(1, 512))

    h = p2[:, : _V2 // 4, :].reshape(2, 625, 8, 64).transpose(
        0, 2, 1, 3).reshape(_B, _FC1IN)
    return _fc_head(h, Wf1, bf1, Wf2, bf2)


# speedup vs baseline: 9.6247x; 1.2993x over previous
import functools

import jax
import jax.numpy as jnp
from jax import lax
from jax.experimental import pallas as pl
from jax.experimental.pallas import tpu as pltpu
from jax.experimental.pallas import tpu_sc as plsc

_B = 16
_D = 10000
_DP = 10240
_V2 = 2500
_V2P = 2560
_K = 25
_F1 = 32
_F2 = 64
_FC1_F = 512
_FC2_F = 10
_FC1IN = 40000

_NCH1, _CH1 = 80, 128
_ET1 = _NCH1 * _CH1
_NCH2, _CH2 = 80, 32
_ET2 = _NCH2 * _CH2

_mesh = plsc.VectorSubcoreMesh(
    core_axis_name="c", subcore_axis_name="s", num_cores=2, num_subcores=16)


def _bcast16(x):
    return jnp.broadcast_to(x, (16,))


_R1 = _DP // 16
_R1C = 128

@functools.partial(
    pl.kernel,
    out_type=jax.ShapeDtypeStruct((_K * _DP, 16), jnp.float32),
    mesh=_mesh,
    compiler_params=pltpu.CompilerParams(use_tc_tiling_on_sc=False),
    scratch_types=[
        pltpu.VMEM_SHARED((_DP, 16), jnp.float32),
        pltpu.VMEM((_NCH1, _CH1), jnp.int32),
        pltpu.VMEM((_NCH1, _CH1), jnp.int32),
        pltpu.VMEM((_ET1 // 16, 16), jnp.float32),
        pltpu.VMEM((_CH1, 16), jnp.float32),
        pltpu.VMEM((_CH1, 16), jnp.float32),
        pltpu.VMEM((_CH1, 16), jnp.float32),
        pltpu.VMEM((_CH1, 16), jnp.float32),
        pltpu.VMEM((_R1C, 16), jnp.float32),
        pltpu.VMEM((_R1C, 16), jnp.float32),
        pltpu.VMEM((_R1C, 16), jnp.float32),
        pltpu.VMEM((_R1C, 16), jnp.float32),
        pltpu.VMEM((16,), jnp.float32),
        pltpu.SemaphoreType.DMA,
        pltpu.SemaphoreType.DMA,
        pltpu.SemaphoreType.DMA,
        pltpu.SemaphoreType.DMA,
        pltpu.SemaphoreType.DMA,
        pltpu.SemaphoreType.DMA,
        pltpu.SemaphoreType.DMA,
        pltpu.SemaphoreType.DMA,
    ],
)
def _sc_l1(x0_hbm, srck_hbm, dst_hbm, val_hbm, s_hbm, out_hbm,
           acc, srci, dsti, vali, gbuf0, gbuf1, gbuf2, gbuf3, aba, abc, abp,
           zbuf, svec, sg0, sg1, sg2, sg3, ss0, ss1, ss2, ss3):
    cid = lax.axis_index("c")
    tid = lax.axis_index("s")

    @pl.when(cid == 0)
    def _body():
        pltpu.sync_copy(dst_hbm.at[tid], dsti)
        pltpu.sync_copy(val_hbm.at[tid], vali)
        pltpu.sync_copy(s_hbm, svec)

        def zrow(i, _):
            zbuf[i, :] = jnp.zeros((16,), jnp.float32)
            return 0
        lax.fori_loop(0, _R1C, zrow, 0)

        def ini(j, _):
            base = tid * _R1 + j * _R1C
            pltpu.sync_copy(x0_hbm.at[pl.ds(base, _R1C)], aba)
            pltpu.sync_copy(aba, out_hbm.at[pl.ds(base, _R1C)])
            pltpu.sync_copy(zbuf, acc.at[pl.ds(base, _R1C)])
            return 0
        lax.fori_loop(0, _R1 // _R1C, ini, 0)
        plsc.subcore_barrier()

        bufs = ((gbuf0, sg0, ss0), (gbuf1, sg1, ss1),
                (gbuf2, sg2, ss2), (gbuf3, sg3, ss3))
        nb = 4

        def g_fire(j, buf, sg):
            pltpu.async_copy(out_hbm.at[srci.at[j]], buf, sg)

        def g_wait(j, buf, sg):
            pltpu.make_async_copy(out_hbm.at[srci.at[j]], buf, sg).wait()

        def s_fire(j, buf, ss):
            pltpu.async_copy(buf, acc.at[dsti.at[j]], ss, add=True)

        def s_wait(j, buf, ss):
            pltpu.make_async_copy(buf, acc.at[dsti.at[j]], ss).wait()

        def hop(k, _):
            pltpu.sync_copy(srck_hbm.at[k, tid], srci)
            for b in range(nb - 1):
                g_fire(b, bufs[b][0], bufs[b][1])

            def quad(g, _):
                for b in range(nb):
                    buf, sg, ss = bufs[b]
                    nbuf, nsg, _nss = bufs[(b + nb - 1) % nb]
                    j = nb * g + b
                    g_wait(j, buf, sg)

                    @pl.when(j + nb - 1 < _NCH1)
                    def _pref():
                        @pl.when(j >= 1)
                        def _dr():
                            s_wait(j - 1, nbuf, _nss)
                        g_fire(j + nb - 1, nbuf, nsg)

                    for gg in range(_CH1 // 16):
                        vrow = vali[j * (_CH1 // 16) + gg, :]
                        for l in range(16):
                            e = gg * 16 + l
                            bc = jnp.take(vrow, jnp.full((16,), l, jnp.int32))
                            buf[e, :] = buf[e, :] * bc
                    s_fire(j, buf, ss)
                return 0
            lax.fori_loop(0, _NCH1 // nb, quad, 0)
            for b in range(nb):
                jj = _NCH1 - nb + b
                s_wait(jj, bufs[jj % nb][0], bufs[jj % nb][2])
            plsc.subcore_barrier()

            w = lax.convert_element_type(k == 0, jnp.float32)
            sv = svec[...]
            alpha = sv * (2.0 - w)
            beta = _bcast16(w - 2.0)
            gamma = _bcast16(w - 1.0)
            pk = jnp.maximum(k - 1, 0)

            def ax(j, _):
                base = tid * _R1 + j * _R1C
                pltpu.sync_copy(acc.at[pl.ds(base, _R1C)], aba)
                pltpu.sync_copy(out_hbm.at[pl.ds(k * _DP + base, _R1C)], abc)
                pltpu.sync_copy(out_hbm.at[pl.ds(pk * _DP + base, _R1C)], abp)

                def row(i, _):
                    abp[i, :] = (alpha * aba[i, :] + beta * abc[i, :]
                                 + gamma * abp[i, :])
                    return 0
                lax.fori_loop(0, _R1C, row, 0)
                pltpu.sync_copy(abp, out_hbm.at[pl.ds((k + 1) * _DP + base, _R1C)])
                pltpu.sync_copy(zbuf, acc.at[pl.ds(base, _R1C)])
                return 0
            lax.fori_loop(0, _R1 // _R1C, ax, 0)
            plsc.subcore_barrier()
            return 0
        lax.fori_loop(0, _K - 1, hop, 0)


_R2 = _V2P // 16
_R2C = 32

@functools.partial(
    pl.kernel,
    out_type=jax.ShapeDtypeStruct((_K * 2 * _V2P, 256), jnp.float32),
    mesh=_mesh,
    compiler_params=pltpu.CompilerParams(use_tc_tiling_on_sc=False),
    scratch_types=[
        pltpu.VMEM_SHARED((_V2P, 256), jnp.float32),
        pltpu.VMEM((_NCH2, _CH2), jnp.int32),
        pltpu.VMEM((_NCH2, _CH2), jnp.int32),
        pltpu.VMEM((_ET2 // 16, 16), jnp.float32),
        pltpu.VMEM((_CH2, 256), jnp.float32),
        pltpu.VMEM((_CH2, 256), jnp.float32),
        pltpu.VMEM((_CH2, 256), jnp.float32),
        pltpu.VMEM((_CH2, 256), jnp.float32),
        pltpu.VMEM((_R2C, 256), jnp.float32),
        pltpu.VMEM((_R2C, 256), jnp.float32),
        pltpu.VMEM((_R2C, 256), jnp.float32),
        pltpu.VMEM((_R2C, 256), jnp.float32),
        pltpu.VMEM((16,), jnp.float32),
        pltpu.SemaphoreType.DMA,
        pltpu.SemaphoreType.DMA,
        pltpu.SemaphoreType.DMA,
        pltpu.SemaphoreType.DMA,
        pltpu.SemaphoreType.DMA,
        pltpu.SemaphoreType.DMA,
        pltpu.SemaphoreType.DMA,
        pltpu.SemaphoreType.DMA,
    ],
)
def _sc_l2(x0_hbm, srck_hbm, dst_hbm, val_hbm, s_hbm, out_hbm,
           acc, srci, dsti, vali, gbuf0, gbuf1, gbuf2, gbuf3, aba, abc, abp,
           zbuf, svec, sg0, sg1, sg2, sg3, ss0, ss1, ss2, ss3):
    cid = lax.axis_index("c")
    tid = lax.axis_index("s")

    pltpu.sync_copy(dst_hbm.at[tid], dsti)
    pltpu.sync_copy(val_hbm.at[tid], vali)
    pltpu.sync_copy(s_hbm, svec)

    def zrow(i, _):
        for r in range(16):
            zbuf[i, pl.ds(r * 16, 16)] = jnp.zeros((16,), jnp.float32)
        return 0
    lax.fori_loop(0, _R2C, zrow, 0)

    def ini(j, _):
        base = tid * _R2 + j * _R2C
        pltpu.sync_copy(x0_hbm.at[cid, pl.ds(base, _R2C)], aba)
        pltpu.sync_copy(aba, out_hbm.at[pl.ds(cid * _V2P + base, _R2C)])
        pltpu.sync_copy(zbuf, acc.at[pl.ds(base, _R2C)])
        return 0
    lax.fori_loop(0, _R2 // _R2C, ini, 0)
    plsc.subcore_barrier()

    bufs = ((gbuf0, sg0, ss0), (gbuf1, sg1, ss1),
            (gbuf2, sg2, ss2), (gbuf3, sg3, ss3))
    nb = 4

    def g_fire(j, buf, sg):
        pltpu.async_copy(out_hbm.at[srci.at[j]], buf, sg)

    def g_wait(j, buf, sg):
        pltpu.make_async_copy(out_hbm.at[srci.at[j]], buf, sg).wait()

    def s_fire(j, buf, ss):
        pltpu.async_copy(buf, acc.at[dsti.at[j]], ss, add=True)

    def s_wait(j, buf, ss):
        pltpu.make_async_copy(buf, acc.at[dsti.at[j]], ss).wait()

    def hop(k, _):
        pltpu.sync_copy(srck_hbm.at[k, cid, tid], srci)
        for b in range(nb - 1):
            g_fire(b, bufs[b][0], bufs[b][1])

        def quad(g, _):
            for b in range(nb):
                buf, sg, ss = bufs[b]
                nbuf, nsg, _nss = bufs[(b + nb - 1) % nb]
                j = nb * g + b
                g_wait(j, buf, sg)

                @pl.when(j + nb - 1 < _NCH2)
                def _pref():
                    @pl.when(j >= 1)
                    def _dr():
                        s_wait(j - 1, nbuf, _nss)
                    g_fire(j + nb - 1, nbuf, nsg)

                def grp(gg, _):
                    vrow = vali[j * (_CH2 // 16) + gg, :]
                    for l in range(16):
                        bc = jnp.take(vrow, jnp.full((16,), l, jnp.int32))
                        for r in range(16):
                            sl = pl.ds(r * 16, 16)
                            buf[gg * 16 + l, sl] = buf[gg * 16 + l, sl] * bc
                    return 0
                lax.fori_loop(0, _CH2 // 16, grp, 0)
                s_fire(j, buf, ss)
            return 0
        lax.fori_loop(0, _NCH2 // nb, quad, 0)
        for b in range(nb):
            jj = _NCH2 - nb + b
            s_wait(jj, bufs[jj % nb][0], bufs[jj % nb][2])
        plsc.subcore_barrier()

        w = lax.convert_element_type(k == 0, jnp.float32)
        sv = svec[...]
        alpha = sv * (2.0 - w)
        beta = _bcast16(w - 2.0)
        gamma = _bcast16(w - 1.0)
        pk = jnp.maximum(k - 1, 0)
        cb = (k * 2 + cid) * _V2P
        pb = (pk * 2 + cid) * _V2P
        nbase = ((k + 1) * 2 + cid) * _V2P

        def ax(j, _):
            base = tid * _R2 + j * _R2C
            pltpu.sync_copy(acc.at[pl.ds(base, _R2C)], aba)
            pltpu.sync_copy(out_hbm.at[pl.ds(cb + base, _R2C)], abc)
            pltpu.sync_copy(out_hbm.at[pl.ds(pb + base, _R2C)], abp)

            def row(i, _):
                for r in range(16):
                    sl = pl.ds(r * 16, 16)
                    abp[i, sl] = (alpha * aba[i, sl] + beta * abc[i, sl]
                                  + gamma * abp[i, sl])
                return 0
            lax.fori_loop(0, _R2C, row, 0)
            pltpu.sync_copy(abp, out_hbm.at[pl.ds(nbase + base, _R2C)])
            pltpu.sync_copy(zbuf, acc.at[pl.ds(base, _R2C)])
            return 0
        lax.fori_loop(0, _R2 // _R2C, ax, 0)
        plsc.subcore_barrier()
        return 0
    lax.fori_loop(0, _K - 1, hop, 0)


def _c1_body(xk_ref, bd_ref, b_ref, o_ref, acc_ref):
    k = pl.program_id(0)

    @pl.when(k == 0)
    def _():
        acc_ref[...] = jnp.zeros_like(acc_ref)

    acc_ref[...] += jax.lax.dot_general(
        xk_ref[0], bd_ref[0], (((1,), (0,)), ((), ())),
        preferred_element_type=jnp.float32)

    @pl.when(k == _K - 1)
    def _():
        h = jnp.maximum(acc_ref[: _D] + b_ref[...], 0.0)
        o_ref[...] = h.reshape(_V2, 4, 512).max(axis=1)


def _contract1(xk1, BD1, bias512):
    return pl.pallas_call(
        _c1_body,
        grid=(_K,),
        in_specs=[
            pl.BlockSpec((1, _DP, 16), lambda k: (k, 0, 0)),
            pl.BlockSpec((1, 16, 512), lambda k: (k, 0, 0)),
            pl.BlockSpec((1, 512), lambda k: (0, 0)),
        ],
        out_specs=pl.BlockSpec((_V2, 512), lambda k: (0, 0)),
        out_shape=jax.ShapeDtypeStruct((_V2, 512), jnp.float32),
        scratch_shapes=[pltpu.VMEM((_DP, 512), jnp.float32)],
    )(xk1, BD1, bias512)


def _c2_body(xk_ref, bd_ref, b_ref, o_ref, acc_ref):
    k = pl.program_id(1)

    @pl.when(k == 0)
    def _():
        acc_ref[...] = jnp.zeros_like(acc_ref)

    acc_ref[...] += jax.lax.dot_general(
        xk_ref[0, 0], bd_ref[0], (((1,), (0,)), ((), ())),
        preferred_element_type=jnp.float32)

    @pl.when(k == _K - 1)
    def _():
        h = jnp.maximum(acc_ref[...] + b_ref[...], 0.0)
        o_ref[0] = h.reshape(_V2P // 4, 4, 512).max(axis=1)


def _contract2(xk2, BD2, bias512):
    return pl.pallas_call(
        _c2_body,
        grid=(2, _K),
        in_specs=[
            pl.BlockSpec((1, 1, _V2P, 256), lambda c, k: (k, c, 0, 0)),
            pl.BlockSpec((1, 256, 512), lambda c, k: (k, 0, 0)),
            pl.BlockSpec((1, 512), lambda c, k: (0, 0)),
        ],
        out_specs=pl.BlockSpec((1, _V2P // 4, 512), lambda c, k: (c, 0, 0)),
        out_shape=jax.ShapeDtypeStruct((2, _V2P // 4, 512), jnp.float32),
        scratch_shapes=[pltpu.VMEM((_V2P, 512), jnp.float32)],
    )(xk2, BD2, bias512)


_FC1_BLK = 128


def _fc1_body(x_ref, w1_ref, b1_ref, o_ref):
    o_ref[...] = jnp.maximum(
        jax.lax.dot_general(
            x_ref[...], w1_ref[...], (((1,), (1,)), ((), ())),
            preferred_element_type=jnp.float32) + b1_ref[...], 0.0)


def _fc2_body(h_ref, w2_ref, b2_ref, o_ref):
    o_ref[...] = jax.lax.dot_general(
        h_ref[...], w2_ref[...], (((1,), (1,)), ((), ())),
        preferred_element_type=jnp.float32) + b2_ref[...]


def _fc_head(h, Wf1, bf1, Wf2, bf2):
    nj = _FC1_F // _FC1_BLK
    h1 = pl.pallas_call(
        _fc1_body,
        grid=(nj,),
        in_specs=[
            pl.BlockSpec((_B, _FC1IN), lambda j: (0, 0)),
            pl.BlockSpec((_FC1_BLK, _FC1IN), lambda j: (j, 0)),
            pl.BlockSpec((1, _FC1_BLK), lambda j: (0, j)),
        ],
        out_specs=pl.BlockSpec((_B, _FC1_BLK), lambda j: (0, j)),
        out_shape=jax.ShapeDtypeStruct((_B, _FC1_F), jnp.float32),
    )(h, Wf1, bf1.reshape(1, -1))
    return pl.pallas_call(
        _fc2_body,
        out_shape=jax.ShapeDtypeStruct((_B, _FC2_F), jnp.float32),
    )(h1, Wf2, bf2.reshape(1, -1))


def _shard_edges(idx, vals, n_tiles, e_per_tile, et_pad, nch, ch, v_mod):
    dst = idx[0].reshape(n_tiles, e_per_tile)
    src = idx[1].reshape(n_tiles, e_per_tile)
    val = vals.reshape(n_tiles, e_per_tile)
    npad = et_pad - e_per_tile
    pad_rows = jnp.broadcast_to(
        (jnp.arange(npad, dtype=jnp.int32) % v_mod), (n_tiles, npad))
    dst = jnp.concatenate([dst, pad_rows], axis=1).reshape(n_tiles, nch, ch)
    src = jnp.concatenate([src, pad_rows], axis=1).reshape(n_tiles, nch, ch)
    val = jnp.concatenate(
        [val, jnp.zeros((n_tiles, npad), jnp.float32)], axis=1
    ).reshape(n_tiles, et_pad // 16, 16)
    return dst, src, val


def kernel(x, d, L0_indices, L0_values, L2_indices, L2_values, lmax, W1, b1,
           W2, b2, Wf1, bf1, Wf2, bf2):
    f32 = jnp.float32
    s0 = (2.0 / lmax[0]).astype(f32)
    s1 = (2.0 / lmax[1]).astype(f32)

    x0_l1 = jnp.pad(x.T, ((0, _DP - _D), (0, 0)))
    dst1, src1, val1 = _shard_edges(
        L0_indices, L0_values, 16, 10000, _ET1, _NCH1, _CH1, _D)
    srck1 = src1[None] + (jnp.arange(_K - 1, dtype=jnp.int32) * _DP)[
        :, None, None, None]
    xk1 = _sc_l1(x0_l1, srck1, dst1, val1, jnp.full((16,), s0, f32))
    xk1 = xk1.reshape(_K, _DP, 16)

    BD1 = jnp.einsum("bc,ko->kbco", jnp.eye(16, dtype=f32),
                     W1.T).reshape(_K, 16, 512)
    p1 = _contract1(xk1, BD1, jnp.tile(b1, 16).reshape(1, 512))

    x0_l2 = jnp.pad(p1, ((0, _V2P - _V2), (0, 0))).reshape(
        _V2P, 2, 256).transpose(1, 0, 2)
    dst2, src2, val2 = _shard_edges(
        L2_indices, L2_values, 16, 2500, _ET2, _NCH2, _CH2, _V2)
    offs = ((jnp.arange(_K - 1, dtype=jnp.int32) * 2)[:, None]
            + jnp.arange(2, dtype=jnp.int32)[None, :]) * _V2P
    srck2 = src2[None, None] + offs[:, :, None, None, None]
    xk2 = _sc_l2(x0_l2, srck2, dst2, val2, jnp.full((16,), s1, f32))
    xk2 = xk2.reshape(_K, 2, _V2P, 256)

    W2r = W2.reshape(_F2, _F1, _K).transpose(2, 1, 0)
    BD2 = jnp.einsum("bc,kfo->kbfco", jnp.eye(8, dtype=f32),
                     W2r).reshape(_K, 256, 512)
    p2 = _contract2(xk2, BD2, jnp.tile(b2, 8).reshape(1, 512))

    h = p2[:, : _V2 // 4, :].reshape(2, 625, 8, 64).transpose(
        0, 2, 1, 3).reshape(_B, _FC1IN)
    return _fc_head(h, Wf1, bf1, Wf2, bf2)
